# baseline (speedup 1.0000x reference)
import functools

import jax
import jax.numpy as jnp
from jax import lax
from jax.experimental import pallas as pl
from jax.experimental.pallas import tpu as pltpu
from jax.experimental.pallas import tpu_sc as plsc

_K = 1024
_B = 64
_N = 32768
_NC, _NS, _L = 2, 16, 16
_NW = _NC * _NS
_RPW = _B // _NW
_NCHUNK = _N // _L
_HB = 4096
_CB = 256
_WBLK = 8192


def _bits_of(xv):
    return lax.bitcast_convert_type(xv, jnp.int32) & jnp.int32(0x7FFFFFFF)


def _splat_total(v):
    return plsc.cummax(lax.rev(plsc.cumsum(v), (0,)))


def _pcnt(m):
    return plsc.all_reduce_population_count(m)


def _scan_chunk(v, base_cum, jvec):
    rv = lax.rev(v, (0,))
    tot = base_cum + plsc.cumsum(rv)
    hit = tot >= _K
    t = plsc.all_reduce_ffs(hit)
    b1 = jvec - t
    above = base_cum + _splat_total(jnp.where(hit, 0, rv))
    return _pcnt(hit) > 0, b1, above, _splat_total(rv)


def _row_select_and_mask(x_hbm, row, row_v, wtmps, wsems, isems, w_hbm,
                         hist_v, chist_v, cb_v, ci_v):
    lanes = lax.iota(jnp.int32, _L)
    ones = jnp.ones((_L,), jnp.int32)
    zsplat = jnp.zeros((_L,), jnp.int32)

    nblk = _N // _WBLK
    xblk = 2 * _WBLK
    handles = [None, None]
    in_handles = [None, None]
    handles[0] = pltpu.async_copy(w_hbm.at[pl.ds(0, _WBLK)], wtmps[0],
                                  wsems[0])
    in_handles[0] = pltpu.async_copy(
        x_hbm.at[row, pl.ds(0, xblk)], row_v.at[pl.ds(0, xblk)], isems[0])

    @plsc.parallel_loop(0, _HB // _L, unroll=8)
    def hz(j):
        hist_v[pl.ds(j * _L, _L)] = jnp.zeros((_L,), jnp.int32)

    for blk in range(nblk):
        handles[blk % 2].wait()
        if blk % 2 == 0:
            xh = blk // 2
            in_handles[xh % 2].wait()
            if xh + 1 < _N // xblk:
                in_handles[(xh + 1) % 2] = pltpu.async_copy(
                    x_hbm.at[row, pl.ds((xh + 1) * xblk, xblk)],
                    row_v.at[pl.ds((xh + 1) * xblk, xblk)],
                    isems[(xh + 1) % 2])
        if blk + 1 < nblk:
            handles[(blk + 1) % 2] = pltpu.async_copy(
                w_hbm.at[pl.ds((blk + 1) * _WBLK, _WBLK)],
                wtmps[(blk + 1) % 2], wsems[(blk + 1) % 2])
        wtmp_v = wtmps[blk % 2]

        @plsc.parallel_loop(0, _WBLK // _L, unroll=8)
        def ha(i, blk=blk, wtmp_v=wtmp_v):
            sl = pl.ds(blk * _WBLK + i * _L, _L)
            xv = row_v[sl] * wtmp_v[pl.ds(i * _L, _L)]
            row_v[sl] = xv
            b = _bits_of(xv)
            plsc.addupdate_scatter(
                hist_v, [lax.shift_right_logical(b, 19)], ones)

    @plsc.parallel_loop(0, _CB // _L, unroll=2)
    def cred(j):
        acc = zsplat
        for c in range(_L):
            tot = _splat_total(hist_v[pl.ds(j * (_L * _L) + c * _L, _L)])
            acc = jnp.where(lanes == c, tot, acc)
        chist_v[pl.ds(j * _L, _L)] = acc

    def csc(jj, carry):
        cum, c1, cabove, jvec = carry
        v = chist_v[pl.ds((_CB // _L - 1 - jj) * _L, _L)]
        hit_any, b1_c, above_c, total = _scan_chunk(v, cum, jvec)
        pred = hit_any & (c1 < 0)
        c1 = jnp.where(pred, b1_c, c1)
        cabove = jnp.where(pred, above_c, cabove)
        return cum + total, c1, cabove, jvec - _L

    _, c1, cabove, _ = lax.fori_loop(
        0, _CB // _L, csc, (zsplat, zsplat - 1, zsplat, zsplat + (_CB - 1)))

    fine = hist_v[pl.ds(c1[0] * _L, _L)]
    _, b1, above, _ = _scan_chunk(fine, cabove, c1 * _L + (_L - 1))
    need = _K - above

    hi_bound = lax.shift_left(b1 + 1, 19)
    lo_bound = lax.shift_left(b1, 19)

    @plsc.parallel_loop(0, _NCHUNK, unroll=8, carry=(zsplat, lanes))
    def collect(i, carry):
        ptr_v, idxv = carry
        sl = pl.ds(i * _L, _L)
        xv = row_v[sl]
        b = _bits_of(xv)
        z0 = b >= hi_bound
        row_v[sl] = jnp.where(z0, 0.0, xv)
        m = (b >= lo_bound) & ~z0
        pos = ptr_v + plsc.cumsum(m.astype(jnp.int32)) - 1
        plsc.store_scatter(cb_v, [pos], b, mask=m)
        plsc.store_scatter(ci_v, [pos], idxv, mask=m)
        return ptr_v + _pcnt(m), idxv + _L

    cnt_v, _ = collect
    nch = lax.div(cnt_v[0] + (_L - 1), _L)

    def cge(mid):
        def csum(c, carry):
            acc, idxv = carry
            v = cb_v[pl.ds(c * _L, _L)]
            ok = (v >= mid) & (idxv < cnt_v)
            return acc + _pcnt(ok), idxv + _L

        return lax.fori_loop(0, nch, csum, (zsplat, lanes))[0]

    def bs_body(_, lohi):
        lo, hi = lohi
        mid = lax.shift_right_logical(lo + hi, 1)
        ge = cge(mid) >= need
        return jnp.where(ge, mid, lo), jnp.where(ge, hi, mid)

    t_thr, _ = lax.fori_loop(
        0, 19, bs_body,
        (lax.shift_left(b1, 19), lax.shift_left(b1 + 1, 19)))

    m_ties = need - cge(t_thr + 1)

    zerosf = jnp.zeros((_L,), jnp.float32)

    def fx_body(c, carry):
        cumeq, idxv = carry
        v = cb_v[pl.ds(c * _L, _L)]
        valid = idxv < cnt_v
        ok = (v == t_thr) & valid
        pref = plsc.cumsum(ok.astype(jnp.int32)) + cumeq
        z = ((v > t_thr) & valid) | (ok & (pref <= m_ties))
        civ = ci_v[pl.ds(c * _L, _L)]
        plsc.store_scatter(row_v, [civ], zerosf, mask=z)
        return cumeq + _pcnt(ok), idxv + _L

    lax.fori_loop(0, nch, fx_body, (zsplat, lanes))


def _sc_kernel(x_hbm, w_hbm, out_hbm, row_v, wtmp0_v, wtmp1_v, hist_v,
               chist_v, cb_v, ci_v, w_spm, wsem0, wsem1, osem0, osem1,
               isem0, isem1):
    sid = lax.axis_index("s")
    wid = sid * _NC + lax.axis_index("c")
    wtmps = (wtmp0_v, wtmp1_v)
    wsems = (wsem0, wsem1)
    osems = (osem0, osem1)
    isems = (isem0, isem1)
    hw = _N // 2
    stripe = _N // _NS
    pltpu.sync_copy(w_hbm.at[pl.ds(sid * stripe, stripe)],
                    w_spm.at[pl.ds(sid * stripe, stripe)])
    plsc.subcore_barrier()
    out_handles = []
    for r in range(_RPW):
        row = wid * _RPW + r
        for h in out_handles:
            h.wait()
        _row_select_and_mask(
            x_hbm, row, row_v, wtmps, wsems, isems, w_spm, hist_v, chist_v,
            cb_v, ci_v)
        out_handles = [
            pltpu.async_copy(row_v, out_hbm.at[row], osems[0])]
    for h in out_handles:
        h.wait()


@functools.partial(jax.jit, donate_argnums=())
def kernel(x, weight):
    mesh = plsc.VectorSubcoreMesh(
        core_axis_name="c", subcore_axis_name="s",
        num_cores=_NC, num_subcores=_NS)
    return pl.kernel(
        _sc_kernel,
        out_type=jax.ShapeDtypeStruct((_B, _N), jnp.float32),
        mesh=mesh,
        compiler_params=pltpu.CompilerParams(needs_layout_passes=False),
        scratch_types=[
            pltpu.VMEM((_N,), jnp.float32),
            pltpu.VMEM((_WBLK,), jnp.float32),
            pltpu.VMEM((_WBLK,), jnp.float32),
            pltpu.VMEM((_HB,), jnp.int32),
            pltpu.VMEM((_CB,), jnp.int32),
            pltpu.VMEM((_N + _L,), jnp.int32),
            pltpu.VMEM((_N + _L,), jnp.int32),
            pltpu.VMEM_SHARED((_N,), jnp.float32),
            pltpu.SemaphoreType.DMA,
            pltpu.SemaphoreType.DMA,
            pltpu.SemaphoreType.DMA,
            pltpu.SemaphoreType.DMA,
            pltpu.SemaphoreType.DMA,
            pltpu.SemaphoreType.DMA,
        ],
    )(x, weight)


# speedup vs baseline: 1.0076x; 1.0076x over previous
import functools

import jax
import jax.numpy as jnp
from jax import lax
from jax.experimental import pallas as pl
from jax.experimental.pallas import tpu as pltpu
from jax.experimental.pallas import tpu_sc as plsc

_K = 1024
_B = 64
_N = 32768
_NC, _NS, _L = 2, 16, 16
_NW = _NC * _NS
_RPW = _B // _NW
_NCHUNK = _N // _L
_HB = 4096
_CB = 256
_WBLK = 8192


def _bits_of(xv):
    return lax.bitcast_convert_type(xv, jnp.int32) & jnp.int32(0x7FFFFFFF)


def _splat_total(v):
    return plsc.cummax(lax.rev(plsc.cumsum(v), (0,)))


def _pcnt(m):
    return plsc.all_reduce_population_count(m)


def _scan_chunk(v, base_cum, jvec):
    rv = lax.rev(v, (0,))
    tot = base_cum + plsc.cumsum(rv)
    hit = tot >= _K
    t = plsc.all_reduce_ffs(hit)
    b1 = jvec - t
    above = base_cum + _splat_total(jnp.where(hit, 0, rv))
    return _pcnt(hit) > 0, b1, above, _splat_total(rv)


def _row_select_and_mask(x_hbm, row, row_v, wtmps, wsems, isems, w_hbm,
                         hist_v, chist_v, cb_v, ci_v):
    lanes = lax.iota(jnp.int32, _L)
    ones = jnp.ones((_L,), jnp.int32)
    zsplat = jnp.zeros((_L,), jnp.int32)

    nblk = _N // _WBLK
    handles = [None, None]
    in_handles = [None, None]
    handles[0] = pltpu.async_copy(w_hbm.at[pl.ds(0, _WBLK)], wtmps[0],
                                  wsems[0])
    in_handles[0] = pltpu.async_copy(
        x_hbm.at[row, pl.ds(0, _WBLK)], row_v.at[pl.ds(0, _WBLK)], isems[0])

    @plsc.parallel_loop(0, _HB // _L, unroll=8)
    def hz(j):
        hist_v[pl.ds(j * _L, _L)] = jnp.zeros((_L,), jnp.int32)

    for blk in range(nblk):
        handles[blk % 2].wait()
        in_handles[blk % 2].wait()
        if blk + 1 < nblk:
            handles[(blk + 1) % 2] = pltpu.async_copy(
                w_hbm.at[pl.ds((blk + 1) * _WBLK, _WBLK)],
                wtmps[(blk + 1) % 2], wsems[(blk + 1) % 2])
            in_handles[(blk + 1) % 2] = pltpu.async_copy(
                x_hbm.at[row, pl.ds((blk + 1) * _WBLK, _WBLK)],
                row_v.at[pl.ds((blk + 1) * _WBLK, _WBLK)],
                isems[(blk + 1) % 2])
        wtmp_v = wtmps[blk % 2]

        @plsc.parallel_loop(0, _WBLK // _L, unroll=8)
        def ha(i, blk=blk, wtmp_v=wtmp_v):
            sl = pl.ds(blk * _WBLK + i * _L, _L)
            xv = row_v[sl] * wtmp_v[pl.ds(i * _L, _L)]
            row_v[sl] = xv
            b = _bits_of(xv)
            plsc.addupdate_scatter(
                hist_v, [lax.shift_right_logical(b, 19)], ones)

    @plsc.parallel_loop(0, _CB // _L, unroll=2)
    def cred(j):
        acc = zsplat
        for c in range(_L):
            tot = _splat_total(hist_v[pl.ds(j * (_L * _L) + c * _L, _L)])
            acc = jnp.where(lanes == c, tot, acc)
        chist_v[pl.ds(j * _L, _L)] = acc

    def csc(jj, carry):
        cum, c1, cabove, jvec = carry
        v = chist_v[pl.ds((_CB // _L - 1 - jj) * _L, _L)]
        hit_any, b1_c, above_c, total = _scan_chunk(v, cum, jvec)
        pred = hit_any & (c1 < 0)
        c1 = jnp.where(pred, b1_c, c1)
        cabove = jnp.where(pred, above_c, cabove)
        return cum + total, c1, cabove, jvec - _L

    _, c1, cabove, _ = lax.fori_loop(
        0, _CB // _L, csc, (zsplat, zsplat - 1, zsplat, zsplat + (_CB - 1)))

    fine = hist_v[pl.ds(c1[0] * _L, _L)]
    _, b1, above, _ = _scan_chunk(fine, cabove, c1 * _L + (_L - 1))
    need = _K - above

    hi_bound = lax.shift_left(b1 + 1, 19)
    lo_bound = lax.shift_left(b1, 19)

    @plsc.parallel_loop(0, _NCHUNK, unroll=8, carry=(zsplat, lanes))
    def collect(i, carry):
        ptr_v, idxv = carry
        sl = pl.ds(i * _L, _L)
        xv = row_v[sl]
        b = _bits_of(xv)
        z0 = b >= hi_bound
        row_v[sl] = jnp.where(z0, 0.0, xv)
        m = (b >= lo_bound) & ~z0
        pos = ptr_v + plsc.cumsum(m.astype(jnp.int32)) - 1
        plsc.store_scatter(cb_v, [pos], b, mask=m)
        plsc.store_scatter(ci_v, [pos], idxv, mask=m)
        return ptr_v + _pcnt(m), idxv + _L

    cnt_v, _ = collect
    nch = lax.div(cnt_v[0] + (_L - 1), _L)

    def cge(mid):
        def csum(c, carry):
            acc, idxv = carry
            v = cb_v[pl.ds(c * _L, _L)]
            ok = (v >= mid) & (idxv < cnt_v)
            return acc + _pcnt(ok), idxv + _L

        return lax.fori_loop(0, nch, csum, (zsplat, lanes))[0]

    def bs_body(_, lohi):
        lo, hi = lohi
        mid = lax.shift_right_logical(lo + hi, 1)
        ge = cge(mid) >= need
        return jnp.where(ge, mid, lo), jnp.where(ge, hi, mid)

    t_thr, _ = lax.fori_loop(
        0, 19, bs_body,
        (lax.shift_left(b1, 19), lax.shift_left(b1 + 1, 19)))

    m_ties = need - cge(t_thr + 1)

    zerosf = jnp.zeros((_L,), jnp.float32)

    def fx_body(c, carry):
        cumeq, idxv = carry
        v = cb_v[pl.ds(c * _L, _L)]
        valid = idxv < cnt_v
        ok = (v == t_thr) & valid
        pref = plsc.cumsum(ok.astype(jnp.int32)) + cumeq
        z = ((v > t_thr) & valid) | (ok & (pref <= m_ties))
        civ = ci_v[pl.ds(c * _L, _L)]
        plsc.store_scatter(row_v, [civ], zerosf, mask=z)
        return cumeq + _pcnt(ok), idxv + _L

    lax.fori_loop(0, nch, fx_body, (zsplat, lanes))


def _sc_kernel(x_hbm, w_hbm, out_hbm, row_v, wtmp0_v, wtmp1_v, hist_v,
               chist_v, cb_v, ci_v, w_spm, wsem0, wsem1, osem0, osem1,
               isem0, isem1):
    sid = lax.axis_index("s")
    wid = sid * _NC + lax.axis_index("c")
    wtmps = (wtmp0_v, wtmp1_v)
    wsems = (wsem0, wsem1)
    osems = (osem0, osem1)
    isems = (isem0, isem1)
    hw = _N // 2
    stripe = _N // _NS
    pltpu.sync_copy(w_hbm.at[pl.ds(sid * stripe, stripe)],
                    w_spm.at[pl.ds(sid * stripe, stripe)])
    plsc.subcore_barrier()
    out_handles = []
    for r in range(_RPW):
        row = wid * _RPW + r
        for h in out_handles:
            h.wait()
        _row_select_and_mask(
            x_hbm, row, row_v, wtmps, wsems, isems, w_spm, hist_v, chist_v,
            cb_v, ci_v)
        out_handles = [
            pltpu.async_copy(
                row_v.at[pl.ds(h * hw, hw)],
                out_hbm.at[row, pl.ds(h * hw, hw)], osems[h])
            for h in range(2)]
    for h in out_handles:
        h.wait()


@functools.partial(jax.jit, donate_argnums=())
def kernel(x, weight):
    mesh = plsc.VectorSubcoreMesh(
        core_axis_name="c", subcore_axis_name="s",
        num_cores=_NC, num_subcores=_NS)
    return pl.kernel(
        _sc_kernel,
        out_type=jax.ShapeDtypeStruct((_B, _N), jnp.float32),
        mesh=mesh,
        compiler_params=pltpu.CompilerParams(needs_layout_passes=False),
        scratch_types=[
            pltpu.VMEM((_N,), jnp.float32),
            pltpu.VMEM((_WBLK,), jnp.float32),
            pltpu.VMEM((_WBLK,), jnp.float32),
            pltpu.VMEM((_HB,), jnp.int32),
            pltpu.VMEM((_CB,), jnp.int32),
            pltpu.VMEM((_N + _L,), jnp.int32),
            pltpu.VMEM((_N + _L,), jnp.int32),
            pltpu.VMEM_SHARED((_N,), jnp.float32),
            pltpu.SemaphoreType.DMA,
            pltpu.SemaphoreType.DMA,
            pltpu.SemaphoreType.DMA,
            pltpu.SemaphoreType.DMA,
            pltpu.SemaphoreType.DMA,
            pltpu.SemaphoreType.DMA,
        ],
    )(x, weight)
